# baseline (speedup 1.0000x reference)
import functools

import jax
import jax.numpy as jnp
from jax.experimental import pallas as pl
from jax.experimental.pallas import tpu as pltpu

_TOPK = 32

_ENC_TB = 512
_ENC_HB = 2048
_MSK_TB = 64
_DEC_TB = 1024
_DEC_KB = 1024


def _encode_kernel(x_ref, w_ref, b_ref, o_ref):
    acc = jax.lax.dot_general(
        x_ref[...], w_ref[...], (((1,), (1,)), ((), ())),
        preferred_element_type=jnp.float32)
    o_ref[...] = jnp.maximum(acc + b_ref[...], 0.0)


def _count_ge(m16):
    n = m16.shape[1]
    while n > 128:
        n //= 2
        m16 = m16[:, :n] + m16[:, n:2 * n]
    return jnp.sum(m16.astype(jnp.int32), axis=1, keepdims=True)


def _threshold_kernel(h_ref, t_ref):
    h = h_ref[...]
    r = h.shape[0]
    bits = jax.lax.bitcast_convert_type(h, jnp.int32)
    hi16 = (bits >> 16).astype(jnp.int16)

    def body1(_, carry):
        lo, hi = carry
        mid = lo + ((hi - lo + 1) >> 1)
        cnt = _count_ge((hi16 >= mid.astype(jnp.int16)).astype(jnp.int16))
        ge = cnt >= _TOPK
        return jnp.where(ge, mid, lo), jnp.where(ge, hi, mid - 1)

    lo1 = jnp.zeros((r, 1), jnp.int32)
    hi1 = jnp.full((r, 1), 32700, jnp.int32)
    p32, _ = jax.lax.fori_loop(0, 15, body1, (lo1, hi1))
    p = p32.astype(jnp.int16)

    c2 = _count_ge((hi16 > p).astype(jnp.int16))
    c = _TOPK - c2
    low_s = jnp.where(hi16 == p,
                      (bits ^ 0x8000).astype(jnp.int16),
                      jnp.int16(-32768))

    def body2(_, carry):
        lo, hi = carry
        mid = lo + ((hi - lo + 1) >> 1)
        cnt = _count_ge((low_s >= mid.astype(jnp.int16)).astype(jnp.int16))
        ge = cnt >= c
        return jnp.where(ge, mid, lo), jnp.where(ge, hi, mid - 1)

    lo2 = jnp.full((r, 1), -32768, jnp.int32)
    hi2 = jnp.full((r, 1), 32767, jnp.int32)
    ls, _ = jax.lax.fori_loop(0, 16, body2, (lo2, hi2))

    t_bits = (p32 << 16) | (ls + 32768)
    t_ref[...] = jnp.where(bits >= t_bits, h, 0.0)


def _decode_kernel(l_ref, w_ref, b_ref, o_ref):
    k = pl.program_id(1)

    @pl.when(k == 0)
    def _():
        o_ref[...] = jnp.broadcast_to(b_ref[...], o_ref.shape)

    o_ref[...] += jax.lax.dot_general(
        l_ref[...].astype(jnp.bfloat16), w_ref[...], (((1,), (1,)), ((), ())),
        preferred_element_type=jnp.float32)


@functools.partial(jax.jit, static_argnames=("interpret",))
def _forward(x, w_enc, b_enc, w_dec, b_dec, interpret=False):
    n, d_in = x.shape
    d_hid = w_enc.shape[0]

    x_bf = x.astype(jnp.bfloat16)
    w_enc_bf = w_enc.astype(jnp.bfloat16)
    w_dec_bf = w_dec.astype(jnp.bfloat16)
    b_enc2 = b_enc.reshape(1, d_hid)
    b_dec2 = b_dec.reshape(1, d_in)

    h = pl.pallas_call(
        _encode_kernel,
        grid=(d_hid // _ENC_HB, n // _ENC_TB),
        in_specs=[
            pl.BlockSpec((_ENC_TB, d_in), lambda hb, tb: (tb, 0)),
            pl.BlockSpec((_ENC_HB, d_in), lambda hb, tb: (hb, 0)),
            pl.BlockSpec((1, _ENC_HB), lambda hb, tb: (0, hb)),
        ],
        out_specs=pl.BlockSpec((_ENC_TB, _ENC_HB), lambda hb, tb: (tb, hb)),
        out_shape=jax.ShapeDtypeStruct((n, d_hid), jnp.float32),
        compiler_params=pltpu.CompilerParams(
            dimension_semantics=("arbitrary", "arbitrary")),
        interpret=interpret,
    )(x_bf, w_enc_bf, b_enc2)

    latent = pl.pallas_call(
        _threshold_kernel,
        grid=(n // _MSK_TB,),
        in_specs=[pl.BlockSpec((_MSK_TB, d_hid), lambda tb: (tb, 0))],
        out_specs=pl.BlockSpec((_MSK_TB, d_hid), lambda tb: (tb, 0)),
        out_shape=jax.ShapeDtypeStruct((n, d_hid), jnp.float32),
        compiler_params=pltpu.CompilerParams(
            dimension_semantics=("arbitrary",)),
        interpret=interpret,
    )(h)

    recon = pl.pallas_call(
        _decode_kernel,
        grid=(n // _DEC_TB, d_hid // _DEC_KB),
        in_specs=[
            pl.BlockSpec((_DEC_TB, _DEC_KB), lambda tb, kb: (tb, kb)),
            pl.BlockSpec((d_in, _DEC_KB), lambda tb, kb: (0, kb)),
            pl.BlockSpec((1, d_in), lambda tb, kb: (0, 0)),
        ],
        out_specs=pl.BlockSpec((_DEC_TB, d_in), lambda tb, kb: (tb, 0)),
        out_shape=jax.ShapeDtypeStruct((n, d_in), jnp.float32),
        compiler_params=pltpu.CompilerParams(
            dimension_semantics=("arbitrary", "arbitrary")),
        interpret=interpret,
    )(latent, w_dec_bf, b_dec2)

    return recon, latent


def kernel(vision_embeddings, W_enc, b_enc, W_dec, b_dec):
    return _forward(vision_embeddings, W_enc, b_enc, W_dec, b_dec)


# speedup vs baseline: 1.0355x; 1.0355x over previous
import functools

import jax
import jax.numpy as jnp
from jax.experimental import pallas as pl
from jax.experimental.pallas import tpu as pltpu

_TOPK = 32

_ENC_TB = 512
_ENC_HB = 2048
_MSK_TB = 128
_DEC_TB = 1024
_DEC_KB = 1024


def _encode_kernel(x_ref, w_ref, b_ref, o_ref):
    acc = jax.lax.dot_general(
        x_ref[...], w_ref[...], (((1,), (1,)), ((), ())),
        preferred_element_type=jnp.float32)
    o_ref[...] = jnp.maximum(acc + b_ref[...], 0.0)


def _count_ge(m16):
    n = m16.shape[1]
    while n > 128:
        n //= 2
        m16 = m16[:, :n] + m16[:, n:2 * n]
    return jnp.sum(m16.astype(jnp.int32), axis=1, keepdims=True)


def _threshold_kernel(h_ref, t_ref):
    h = h_ref[...]
    r = h.shape[0]
    bits = jax.lax.bitcast_convert_type(h, jnp.int32)
    hi16 = (bits >> 16).astype(jnp.int16)

    def body1(_, carry):
        lo, hi = carry
        mid = lo + ((hi - lo + 1) >> 1)
        cnt = _count_ge((hi16 >= mid.astype(jnp.int16)).astype(jnp.int16))
        ge = cnt >= _TOPK
        return jnp.where(ge, mid, lo), jnp.where(ge, hi, mid - 1)

    lo1 = jnp.zeros((r, 1), jnp.int32)
    hi1 = jnp.full((r, 1), 32700, jnp.int32)
    p32, _ = jax.lax.fori_loop(0, 15, body1, (lo1, hi1))
    p = p32.astype(jnp.int16)

    c2 = _count_ge((hi16 > p).astype(jnp.int16))
    c = _TOPK - c2
    low_s = jnp.where(hi16 == p,
                      (bits ^ 0x8000).astype(jnp.int16),
                      jnp.int16(-32768))

    def body2(_, carry):
        lo, hi = carry
        mid = lo + ((hi - lo + 1) >> 1)
        cnt = _count_ge((low_s >= mid.astype(jnp.int16)).astype(jnp.int16))
        ge = cnt >= c
        return jnp.where(ge, mid, lo), jnp.where(ge, hi, mid - 1)

    lo2 = jnp.full((r, 1), -32768, jnp.int32)
    hi2 = jnp.full((r, 1), 32767, jnp.int32)
    ls, _ = jax.lax.fori_loop(0, 16, body2, (lo2, hi2))

    t_bits = (p32 << 16) | (ls + 32768)
    t_ref[...] = jnp.where(bits >= t_bits, h, 0.0)


def _decode_kernel(l_ref, w_ref, b_ref, o_ref):
    k = pl.program_id(1)

    @pl.when(k == 0)
    def _():
        o_ref[...] = jnp.broadcast_to(b_ref[...], o_ref.shape)

    o_ref[...] += jax.lax.dot_general(
        l_ref[...].astype(jnp.bfloat16), w_ref[...], (((1,), (1,)), ((), ())),
        preferred_element_type=jnp.float32)


@functools.partial(jax.jit, static_argnames=("interpret",))
def _forward(x, w_enc, b_enc, w_dec, b_dec, interpret=False):
    n, d_in = x.shape
    d_hid = w_enc.shape[0]

    x_bf = x.astype(jnp.bfloat16)
    w_enc_bf = w_enc.astype(jnp.bfloat16)
    w_dec_bf = w_dec.astype(jnp.bfloat16)
    b_enc2 = b_enc.reshape(1, d_hid)
    b_dec2 = b_dec.reshape(1, d_in)

    h = pl.pallas_call(
        _encode_kernel,
        grid=(d_hid // _ENC_HB, n // _ENC_TB),
        in_specs=[
            pl.BlockSpec((_ENC_TB, d_in), lambda hb, tb: (tb, 0)),
            pl.BlockSpec((_ENC_HB, d_in), lambda hb, tb: (hb, 0)),
            pl.BlockSpec((1, _ENC_HB), lambda hb, tb: (0, hb)),
        ],
        out_specs=pl.BlockSpec((_ENC_TB, _ENC_HB), lambda hb, tb: (tb, hb)),
        out_shape=jax.ShapeDtypeStruct((n, d_hid), jnp.float32),
        compiler_params=pltpu.CompilerParams(
            dimension_semantics=("arbitrary", "arbitrary")),
        interpret=interpret,
    )(x_bf, w_enc_bf, b_enc2)

    latent = pl.pallas_call(
        _threshold_kernel,
        grid=(n // _MSK_TB,),
        in_specs=[pl.BlockSpec((_MSK_TB, d_hid), lambda tb: (tb, 0))],
        out_specs=pl.BlockSpec((_MSK_TB, d_hid), lambda tb: (tb, 0)),
        out_shape=jax.ShapeDtypeStruct((n, d_hid), jnp.float32),
        compiler_params=pltpu.CompilerParams(
            dimension_semantics=("arbitrary",)),
        interpret=interpret,
    )(h)

    recon = pl.pallas_call(
        _decode_kernel,
        grid=(n // _DEC_TB, d_hid // _DEC_KB),
        in_specs=[
            pl.BlockSpec((_DEC_TB, _DEC_KB), lambda tb, kb: (tb, kb)),
            pl.BlockSpec((d_in, _DEC_KB), lambda tb, kb: (0, kb)),
            pl.BlockSpec((1, d_in), lambda tb, kb: (0, 0)),
        ],
        out_specs=pl.BlockSpec((_DEC_TB, d_in), lambda tb, kb: (tb, 0)),
        out_shape=jax.ShapeDtypeStruct((n, d_in), jnp.float32),
        compiler_params=pltpu.CompilerParams(
            dimension_semantics=("arbitrary", "arbitrary")),
        interpret=interpret,
    )(latent, w_dec_bf, b_dec2)

    return recon, latent


def kernel(vision_embeddings, W_enc, b_enc, W_dec, b_dec):
    return _forward(vision_embeddings, W_enc, b_enc, W_dec, b_dec)
